# baseline (speedup 1.0000x reference)
import functools

import jax
import jax.numpy as jnp
from jax import lax
from jax.experimental import pallas as pl
from jax.experimental.pallas import tpu as pltpu
from jax.experimental.pallas import tpu_sc as plsc

NC = 2
NS = 16
NW = NC * NS
BLK = 128


def _deg_call(nblk, n_pad):
  mesh = plsc.VectorSubcoreMesh(
      core_axis_name="c", subcore_axis_name="s", num_cores=NC,
      num_subcores=NS)
  rpt = n_pad // NS

  @functools.partial(
      pl.kernel,
      out_type=jax.ShapeDtypeStruct((NC, 2, n_pad), jnp.float32),
      mesh=mesh,
      scratch_types=[
          pltpu.VMEM((nblk, BLK), jnp.int32),
          pltpu.VMEM((nblk, BLK), jnp.int32),
          pltpu.VMEM((BLK,), jnp.float32),
          pltpu.VMEM_SHARED((n_pad,), jnp.float32),
          pltpu.VMEM_SHARED((n_pad,), jnp.float32),
          pltpu.SemaphoreType.DMA,
          pltpu.SemaphoreType.DMA,
      ],
  )
  def deg_k(src_hbm, dst_hbm, zeros_hbm, out_hbm, sidx, didx, ones_v,
            deg_s, deg_d, sem_a, sem_b):
    c = lax.axis_index("c")
    s = lax.axis_index("s")
    wid = c * NS + s
    for i in range(BLK // 16):
      ones_v[pl.ds(16 * i, 16)] = jnp.ones((16,), jnp.float32)
    pltpu.sync_copy(zeros_hbm.at[pl.ds(s * rpt, rpt)],
                    deg_s.at[pl.ds(s * rpt, rpt)])
    pltpu.sync_copy(zeros_hbm.at[pl.ds(s * rpt, rpt)],
                    deg_d.at[pl.ds(s * rpt, rpt)])
    pltpu.sync_copy(src_hbm.at[wid], sidx)
    pltpu.sync_copy(dst_hbm.at[wid], didx)
    plsc.subcore_barrier()

    def body(b, carry):
      ca = pltpu.async_copy(ones_v, deg_s.at[sidx.at[b]], sem_a, add=True)
      cb = pltpu.async_copy(ones_v, deg_d.at[didx.at[b]], sem_b, add=True)
      ca.wait()
      cb.wait()
      return carry

    lax.fori_loop(0, nblk, body, 0)
    plsc.subcore_barrier()
    pltpu.sync_copy(deg_s.at[pl.ds(s * rpt, rpt)],
                    out_hbm.at[c, 0, pl.ds(s * rpt, rpt)])
    pltpu.sync_copy(deg_d.at[pl.ds(s * rpt, rpt)],
                    out_hbm.at[c, 1, pl.ds(s * rpt, rpt)])

  return deg_k


def _agg_call(nblk, n_pad, d):
  mesh = plsc.VectorSubcoreMesh(
      core_axis_name="c", subcore_axis_name="s", num_cores=NC,
      num_subcores=NS)
  rpt = n_pad // NS

  @functools.partial(
      pl.kernel,
      out_type=jax.ShapeDtypeStruct((NC, n_pad, d), jnp.float32),
      mesh=mesh,
      scratch_types=[
          pltpu.VMEM((nblk, BLK), jnp.int32),
          pltpu.VMEM((nblk, BLK), jnp.int32),
          pltpu.VMEM((BLK, d), jnp.float32),
          pltpu.VMEM_SHARED((n_pad, d), jnp.float32),
          pltpu.SemaphoreType.DMA,
      ],
  )
  def agg_k(x_hbm, src_hbm, dst_hbm, z_hbm, out_hbm, sidx, didx, rows,
            agg_sh, sem):
    c = lax.axis_index("c")
    s = lax.axis_index("s")
    wid = c * NS + s
    for j in range(rpt // BLK):
      pltpu.sync_copy(z_hbm,
                      agg_sh.at[pl.ds((s * (rpt // BLK) + j) * BLK, BLK)])
    pltpu.sync_copy(src_hbm.at[wid], sidx)
    pltpu.sync_copy(dst_hbm.at[wid], didx)
    plsc.subcore_barrier()

    def body(b, carry):
      pltpu.async_copy(x_hbm.at[sidx.at[b]], rows, sem).wait()
      pltpu.sync_copy(rows, agg_sh.at[didx.at[b]], add=True)
      return carry

    lax.fori_loop(0, nblk, body, 0)
    plsc.subcore_barrier()
    pltpu.sync_copy(agg_sh.at[pl.ds(s * rpt, rpt)],
                    out_hbm.at[c, pl.ds(s * rpt, rpt)])

  return agg_k


def _xnorm_body(deg_ref, h_ref, x_ref):
  deg = deg_ref[0, 0] + deg_ref[1, 0]
  norm = lax.rsqrt(jnp.maximum(deg, 1.0))
  x_ref[...] = h_ref[...] * norm[:, None]


def _final_body(parts_ref, deg_ref, w_ref, b_ref, a_ref, hout_ref,
                alpha_ref):
  deg = deg_ref[0, 1] + deg_ref[1, 1]
  norm = lax.rsqrt(jnp.maximum(deg, 1.0))
  agg = (parts_ref[0] + parts_ref[1]) * norm[:, None]
  out = jnp.dot(agg, w_ref[...], preferred_element_type=jnp.float32,
                precision=lax.Precision.HIGHEST) + b_ref[...][None, :]
  t = jnp.sum(out * a_ref[...][:, 0][None, :], axis=1, keepdims=True)
  alpha = jax.nn.sigmoid(t)
  hout_ref[...] = out * alpha
  alpha_ref[...] = alpha


def kernel(h, edge_index, W, b, a):
  n, d_in = h.shape
  d_out = W.shape[1]
  e = edge_index.shape[1]
  nblk = -(-e // (NW * BLK))
  nblk = -(-nblk // 16) * 16
  e_pad = nblk * NW * BLK
  n_pad = -(-(n + 1) // (NS * BLK)) * (NS * BLK)
  pad = e_pad - e
  src_p = jnp.concatenate(
      [edge_index[0], jnp.full((pad,), n, jnp.int32)]).reshape(NW, nblk, BLK)
  dst_p = jnp.concatenate(
      [edge_index[1], jnp.full((pad,), n, jnp.int32)]).reshape(NW, nblk, BLK)
  zdeg = jnp.zeros((n_pad,), jnp.float32)
  zrow = jnp.zeros((BLK, d_in), jnp.float32)

  deg_parts = _deg_call(nblk, n_pad)(src_p, dst_p, zdeg)

  grid = n_pad // 1024
  x = pl.pallas_call(
      _xnorm_body,
      grid=(grid,),
      in_specs=[
          pl.BlockSpec((NC, 2, 1024), lambda i: (0, 0, i)),
          pl.BlockSpec((1024, d_in), lambda i: (i, 0)),
      ],
      out_specs=pl.BlockSpec((1024, d_in), lambda i: (i, 0)),
      out_shape=jax.ShapeDtypeStruct((n_pad, d_in), jnp.float32),
  )(deg_parts, h)

  parts = _agg_call(nblk, n_pad, d_in)(x, src_p, dst_p, zrow)

  h_out, alpha = pl.pallas_call(
      _final_body,
      grid=(grid,),
      in_specs=[
          pl.BlockSpec((NC, 1024, d_in), lambda i: (0, i, 0)),
          pl.BlockSpec((NC, 2, 1024), lambda i: (0, 0, i)),
          pl.BlockSpec((d_in, d_out), lambda i: (0, 0)),
          pl.BlockSpec((d_out,), lambda i: (0,)),
          pl.BlockSpec((d_out, 1), lambda i: (0, 0)),
      ],
      out_specs=[
          pl.BlockSpec((1024, d_out), lambda i: (i, 0)),
          pl.BlockSpec((1024, 1), lambda i: (i, 0)),
      ],
      out_shape=[
          jax.ShapeDtypeStruct((n, d_out), jnp.float32),
          jax.ShapeDtypeStruct((n, 1), jnp.float32),
      ],
  )(parts, deg_parts, W, b, a)

  return (h_out, alpha)


# speedup vs baseline: 2.6606x; 2.6606x over previous
import functools

import jax
import jax.numpy as jnp
from jax import lax
from jax.experimental import pallas as pl
from jax.experimental.pallas import tpu as pltpu
from jax.experimental.pallas import tpu_sc as plsc

NC = 2
NS = 16
NW = NC * NS
BLK = 128


def _deg_call(nfull, t8, n_pad):
  mesh = plsc.VectorSubcoreMesh(
      core_axis_name="c", subcore_axis_name="s", num_cores=NC,
      num_subcores=NS)
  rpt = n_pad // NS

  @functools.partial(
      pl.kernel,
      out_type=jax.ShapeDtypeStruct((NC, 2, n_pad), jnp.float32),
      mesh=mesh,
      scratch_types=[
          pltpu.VMEM((nfull, BLK), jnp.int32),
          pltpu.VMEM((nfull, BLK), jnp.int32),
          pltpu.VMEM((max(t8, 8),), jnp.int32),
          pltpu.VMEM((max(t8, 8),), jnp.int32),
          pltpu.VMEM((BLK,), jnp.float32),
          pltpu.VMEM_SHARED((n_pad,), jnp.float32),
          pltpu.VMEM_SHARED((n_pad,), jnp.float32),
          pltpu.SemaphoreType.DMA,
          pltpu.SemaphoreType.DMA,
      ],
  )
  def deg_k(src_hbm, dst_hbm, tsrc_hbm, tdst_hbm, zeros_hbm, out_hbm,
            sidx, didx, tsidx, tdidx, ones_v, deg_s, deg_d, sem_a, sem_b):
    c = lax.axis_index("c")
    s = lax.axis_index("s")
    wid = c * NS + s
    for i in range(BLK // 16):
      ones_v[pl.ds(16 * i, 16)] = jnp.ones((16,), jnp.float32)
    pltpu.sync_copy(zeros_hbm.at[pl.ds(s * rpt, rpt)],
                    deg_s.at[pl.ds(s * rpt, rpt)])
    pltpu.sync_copy(zeros_hbm.at[pl.ds(s * rpt, rpt)],
                    deg_d.at[pl.ds(s * rpt, rpt)])
    pltpu.sync_copy(src_hbm.at[wid], sidx)
    pltpu.sync_copy(dst_hbm.at[wid], didx)
    if t8:
      pltpu.sync_copy(tsrc_hbm.at[wid], tsidx)
      pltpu.sync_copy(tdst_hbm.at[wid], tdidx)
    plsc.subcore_barrier()

    def body(b, carry):
      ca = pltpu.async_copy(ones_v, deg_s.at[sidx.at[b]], sem_a, add=True)
      cb = pltpu.async_copy(ones_v, deg_d.at[didx.at[b]], sem_b, add=True)
      ca.wait()
      cb.wait()
      return carry

    lax.fori_loop(0, nfull, body, 0)
    if t8:
      ca = pltpu.async_copy(ones_v.at[pl.ds(0, t8)], deg_s.at[tsidx],
                            sem_a, add=True)
      cb = pltpu.async_copy(ones_v.at[pl.ds(0, t8)], deg_d.at[tdidx],
                            sem_b, add=True)
      ca.wait()
      cb.wait()
    plsc.subcore_barrier()
    pltpu.sync_copy(deg_s.at[pl.ds(s * rpt, rpt)],
                    out_hbm.at[c, 0, pl.ds(s * rpt, rpt)])
    pltpu.sync_copy(deg_d.at[pl.ds(s * rpt, rpt)],
                    out_hbm.at[c, 1, pl.ds(s * rpt, rpt)])

  return deg_k


def _agg_call(nfull, t8, n_pad, d):
  mesh = plsc.VectorSubcoreMesh(
      core_axis_name="c", subcore_axis_name="s", num_cores=NC,
      num_subcores=NS)
  rpt = n_pad // NS

  @functools.partial(
      pl.kernel,
      out_type=jax.ShapeDtypeStruct((NC, n_pad, d), jnp.float32),
      mesh=mesh,
      scratch_types=[
          pltpu.VMEM((nfull, BLK), jnp.int32),
          pltpu.VMEM((nfull, BLK), jnp.int32),
          pltpu.VMEM((max(t8, 8),), jnp.int32),
          pltpu.VMEM((max(t8, 8),), jnp.int32),
          pltpu.VMEM((BLK, d), jnp.float32),
          pltpu.VMEM_SHARED((n_pad, d), jnp.float32),
          pltpu.SemaphoreType.DMA,
      ],
  )
  def agg_k(x_hbm, src_hbm, dst_hbm, tsrc_hbm, tdst_hbm, z_hbm, out_hbm,
            sidx, didx, tsidx, tdidx, rows, agg_sh, sem):
    c = lax.axis_index("c")
    s = lax.axis_index("s")
    wid = c * NS + s
    for j in range(rpt // BLK):
      pltpu.sync_copy(z_hbm,
                      agg_sh.at[pl.ds((s * (rpt // BLK) + j) * BLK, BLK)])
    pltpu.sync_copy(src_hbm.at[wid], sidx)
    pltpu.sync_copy(dst_hbm.at[wid], didx)
    if t8:
      pltpu.sync_copy(tsrc_hbm.at[wid], tsidx)
      pltpu.sync_copy(tdst_hbm.at[wid], tdidx)
    plsc.subcore_barrier()

    def body(b, carry):
      pltpu.async_copy(x_hbm.at[sidx.at[b]], rows, sem).wait()
      pltpu.sync_copy(rows, agg_sh.at[didx.at[b]], add=True)
      return carry

    lax.fori_loop(0, nfull, body, 0)
    if t8:
      pltpu.async_copy(x_hbm.at[tsidx], rows.at[pl.ds(0, t8)], sem).wait()
      pltpu.sync_copy(rows.at[pl.ds(0, t8)], agg_sh.at[tdidx], add=True)
    plsc.subcore_barrier()
    pltpu.sync_copy(agg_sh.at[pl.ds(s * rpt, rpt)],
                    out_hbm.at[c, pl.ds(s * rpt, rpt)])

  return agg_k


def _xnorm_body(deg_ref, h_ref, x_ref):
  deg = deg_ref[0, 0] + deg_ref[1, 0]
  norm = lax.rsqrt(jnp.maximum(deg, 1.0))
  x_ref[...] = h_ref[...] * norm[:, None]


def _final_body(parts_ref, deg_ref, w_ref, b_ref, a_ref, hout_ref,
                alpha_ref):
  deg = deg_ref[0, 1] + deg_ref[1, 1]
  norm = lax.rsqrt(jnp.maximum(deg, 1.0))
  agg = (parts_ref[0] + parts_ref[1]) * norm[:, None]
  out = jnp.dot(agg, w_ref[...], preferred_element_type=jnp.float32,
                precision=lax.Precision.HIGHEST) + b_ref[...][None, :]
  t = jnp.sum(out * a_ref[...][:, 0][None, :], axis=1, keepdims=True)
  alpha = jax.nn.sigmoid(t)
  hout_ref[...] = out * alpha
  alpha_ref[...] = alpha


def kernel(h, edge_index, W, b, a):
  n, d_in = h.shape
  d_out = W.shape[1]
  e = edge_index.shape[1]
  n_pad = -(-(n + 1) // (NS * BLK)) * (NS * BLK)

  ept = -(-e // NW)
  nfull = ept // BLK
  t = ept - nfull * BLK
  t8 = -(-t // 8) * 8
  cap = nfull * BLK + t8
  fill = NW * cap - e

  spare = jnp.arange(fill, dtype=jnp.int32) % (n_pad - n - 1)
  src_f = jnp.concatenate([edge_index[0], jnp.full((fill,), n, jnp.int32)])
  dst_f = jnp.concatenate([edge_index[1], n + 1 + spare])
  src_f = src_f.reshape(NW, cap)
  dst_f = dst_f.reshape(NW, cap)
  src_m = src_f[:, :nfull * BLK].reshape(NW, nfull, BLK)
  dst_m = dst_f[:, :nfull * BLK].reshape(NW, nfull, BLK)
  if t8:
    src_t = src_f[:, nfull * BLK:]
    dst_t = dst_f[:, nfull * BLK:]
  else:
    src_t = jnp.zeros((NW, 8), jnp.int32)
    dst_t = jnp.full((NW, 8), n, jnp.int32)
  zdeg = jnp.zeros((n_pad,), jnp.float32)
  zrow = jnp.zeros((BLK, d_in), jnp.float32)

  deg_parts = _deg_call(nfull, t8, n_pad)(
      src_m, dst_m, src_t, dst_t, zdeg)

  grid = n_pad // 1024
  x = pl.pallas_call(
      _xnorm_body,
      grid=(grid,),
      in_specs=[
          pl.BlockSpec((NC, 2, 1024), lambda i: (0, 0, i)),
          pl.BlockSpec((1024, d_in), lambda i: (i, 0)),
      ],
      out_specs=pl.BlockSpec((1024, d_in), lambda i: (i, 0)),
      out_shape=jax.ShapeDtypeStruct((n_pad, d_in), jnp.float32),
  )(deg_parts, h)

  parts = _agg_call(nfull, t8, n_pad, d_in)(
      x, src_m, dst_m, src_t, dst_t, zrow)

  h_out, alpha = pl.pallas_call(
      _final_body,
      grid=(grid,),
      in_specs=[
          pl.BlockSpec((NC, 1024, d_in), lambda i: (0, i, 0)),
          pl.BlockSpec((NC, 2, 1024), lambda i: (0, 0, i)),
          pl.BlockSpec((d_in, d_out), lambda i: (0, 0)),
          pl.BlockSpec((d_out,), lambda i: (0,)),
          pl.BlockSpec((d_out, 1), lambda i: (0, 0)),
      ],
      out_specs=[
          pl.BlockSpec((1024, d_out), lambda i: (i, 0)),
          pl.BlockSpec((1024, 1), lambda i: (i, 0)),
      ],
      out_shape=[
          jax.ShapeDtypeStruct((n, d_out), jnp.float32),
          jax.ShapeDtypeStruct((n, 1), jnp.float32),
      ],
  )(parts, deg_parts, W, b, a)

  return (h_out, alpha)


# speedup vs baseline: 3.1015x; 1.1657x over previous
import functools

import jax
import jax.numpy as jnp
from jax import lax
from jax.experimental import pallas as pl
from jax.experimental.pallas import tpu as pltpu
from jax.experimental.pallas import tpu_sc as plsc

NC = 2
NS = 16
NW = NC * NS
BLK = 128


def _deg_call(nfull, t8, n_pad):
  mesh = plsc.VectorSubcoreMesh(
      core_axis_name="c", subcore_axis_name="s", num_cores=NC,
      num_subcores=NS)
  rpt = n_pad // NS

  @functools.partial(
      pl.kernel,
      out_type=jax.ShapeDtypeStruct((NC, 2, n_pad), jnp.float32),
      mesh=mesh,
      scratch_types=[
          pltpu.VMEM((nfull, BLK), jnp.int32),
          pltpu.VMEM((nfull, BLK), jnp.int32),
          pltpu.VMEM((max(t8, 8),), jnp.int32),
          pltpu.VMEM((max(t8, 8),), jnp.int32),
          pltpu.VMEM((BLK,), jnp.float32),
          pltpu.VMEM_SHARED((n_pad,), jnp.float32),
          pltpu.VMEM_SHARED((n_pad,), jnp.float32),
          pltpu.SemaphoreType.DMA,
          pltpu.SemaphoreType.DMA,
      ],
  )
  def deg_k(src_hbm, dst_hbm, tsrc_hbm, tdst_hbm, zeros_hbm, out_hbm,
            sidx, didx, tsidx, tdidx, ones_v, deg_s, deg_d, sem_a, sem_b):
    c = lax.axis_index("c")
    s = lax.axis_index("s")
    wid = c * NS + s
    for i in range(BLK // 16):
      ones_v[pl.ds(16 * i, 16)] = jnp.ones((16,), jnp.float32)
    pltpu.sync_copy(zeros_hbm.at[pl.ds(s * rpt, rpt)],
                    deg_s.at[pl.ds(s * rpt, rpt)])
    pltpu.sync_copy(zeros_hbm.at[pl.ds(s * rpt, rpt)],
                    deg_d.at[pl.ds(s * rpt, rpt)])
    pltpu.sync_copy(src_hbm.at[wid], sidx)
    pltpu.sync_copy(dst_hbm.at[wid], didx)
    if t8:
      pltpu.sync_copy(tsrc_hbm.at[wid], tsidx)
      pltpu.sync_copy(tdst_hbm.at[wid], tdidx)
    plsc.subcore_barrier()

    def body(b, carry):
      ca = pltpu.async_copy(ones_v, deg_s.at[sidx.at[b]], sem_a, add=True)
      cb = pltpu.async_copy(ones_v, deg_d.at[didx.at[b]], sem_b, add=True)
      ca.wait()
      cb.wait()
      return carry

    lax.fori_loop(0, nfull, body, 0)
    if t8:
      ca = pltpu.async_copy(ones_v.at[pl.ds(0, t8)], deg_s.at[tsidx],
                            sem_a, add=True)
      cb = pltpu.async_copy(ones_v.at[pl.ds(0, t8)], deg_d.at[tdidx],
                            sem_b, add=True)
      ca.wait()
      cb.wait()
    plsc.subcore_barrier()
    pltpu.sync_copy(deg_s.at[pl.ds(s * rpt, rpt)],
                    out_hbm.at[c, 0, pl.ds(s * rpt, rpt)])
    pltpu.sync_copy(deg_d.at[pl.ds(s * rpt, rpt)],
                    out_hbm.at[c, 1, pl.ds(s * rpt, rpt)])

  return deg_k


def _agg_call(nfull, t8, n_pad, d):
  mesh = plsc.VectorSubcoreMesh(
      core_axis_name="c", subcore_axis_name="s", num_cores=NC,
      num_subcores=NS)
  rpt = n_pad // NS

  sb = 2
  for cand in range(2, 33, 2):
    if nfull % cand == 0:
      sb = cand
  pipelined = nfull % 2 == 0 and nfull > 0

  @functools.partial(
      pl.kernel,
      out_type=jax.ShapeDtypeStruct((NC, n_pad, d), jnp.float32),
      mesh=mesh,
      scratch_types=[
          pltpu.VMEM((sb if pipelined else max(nfull, 1), BLK), jnp.int32),
          pltpu.VMEM((sb if pipelined else max(nfull, 1), BLK), jnp.int32),
          pltpu.VMEM((max(t8, 8),), jnp.int32),
          pltpu.VMEM((max(t8, 8),), jnp.int32),
          pltpu.VMEM((BLK, d), jnp.float32),
          pltpu.VMEM((BLK, d), jnp.float32),
          pltpu.VMEM_SHARED((n_pad, d), jnp.float32),
          pltpu.SemaphoreType.DMA,
          pltpu.SemaphoreType.DMA,
          pltpu.SemaphoreType.DMA,
          pltpu.SemaphoreType.DMA,
      ],
  )
  def agg_k(x_hbm, src_hbm, dst_hbm, tsrc_hbm, tdst_hbm, z_hbm, out_hbm,
            sidx, didx, tsidx, tdidx, r0, r1, agg_sh, sg0, sg1, ss0, ss1):
    c = lax.axis_index("c")
    s = lax.axis_index("s")
    wid = c * NS + s
    for j in range(rpt // BLK):
      pltpu.sync_copy(z_hbm,
                      agg_sh.at[pl.ds((s * (rpt // BLK) + j) * BLK, BLK)])
    if t8:
      pltpu.sync_copy(tsrc_hbm.at[wid], tsidx)
      pltpu.sync_copy(tdst_hbm.at[wid], tdidx)
    plsc.subcore_barrier()

    if pipelined:
      def sb_body(g, carry):
        pltpu.sync_copy(src_hbm.at[wid, g], sidx)
        pltpu.sync_copy(dst_hbm.at[wid, g], didx)
        pltpu.async_copy(x_hbm.at[sidx.at[0]], r0, sg0)

        def body(k, c2):
          pltpu.make_async_copy(x_hbm.at[sidx.at[0]], r0, sg0).wait()

          @pl.when(k > 0)
          def _():
            pltpu.make_async_copy(r1, agg_sh.at[didx.at[0]], ss1).wait()

          pltpu.async_copy(x_hbm.at[sidx.at[2 * k + 1]], r1, sg1)
          pltpu.async_copy(r0, agg_sh.at[didx.at[2 * k]], ss0, add=True)
          pltpu.make_async_copy(x_hbm.at[sidx.at[0]], r1, sg1).wait()
          pltpu.make_async_copy(r0, agg_sh.at[didx.at[0]], ss0).wait()

          @pl.when(k < sb // 2 - 1)
          def _():
            pltpu.async_copy(x_hbm.at[sidx.at[2 * k + 2]], r0, sg0)

          pltpu.async_copy(r1, agg_sh.at[didx.at[2 * k + 1]], ss1,
                           add=True)
          return c2

        lax.fori_loop(0, sb // 2, body, 0)
        pltpu.make_async_copy(r1, agg_sh.at[didx.at[0]], ss1).wait()
        return carry

      lax.fori_loop(0, nfull // sb, sb_body, 0)
    elif nfull:
      pltpu.sync_copy(src_hbm.at[wid, 0], sidx)
      pltpu.sync_copy(dst_hbm.at[wid, 0], didx)

      def body(b, carry):
        pltpu.async_copy(x_hbm.at[sidx.at[b]], r0, sg0).wait()
        pltpu.sync_copy(r0, agg_sh.at[didx.at[b]], add=True)
        return carry

      lax.fori_loop(0, nfull, body, 0)
    if t8:
      pltpu.async_copy(x_hbm.at[tsidx], r0.at[pl.ds(0, t8)], sg0).wait()
      pltpu.sync_copy(r0.at[pl.ds(0, t8)], agg_sh.at[tdidx], add=True)
    plsc.subcore_barrier()
    pltpu.sync_copy(agg_sh.at[pl.ds(s * rpt, rpt)],
                    out_hbm.at[c, pl.ds(s * rpt, rpt)])

  return agg_k


def _xnorm_body(deg_ref, h_ref, x_ref):
  deg = deg_ref[0, 0] + deg_ref[1, 0]
  norm = lax.rsqrt(jnp.maximum(deg, 1.0))
  x_ref[...] = h_ref[...] * norm[:, None]


def _final_body(parts_ref, deg_ref, w_ref, b_ref, a_ref, hout_ref,
                alpha_ref):
  deg = deg_ref[0, 1] + deg_ref[1, 1]
  norm = lax.rsqrt(jnp.maximum(deg, 1.0))
  agg = (parts_ref[0] + parts_ref[1]) * norm[:, None]
  out = jnp.dot(agg, w_ref[...], preferred_element_type=jnp.float32,
                precision=lax.Precision.HIGHEST) + b_ref[...][None, :]
  t = jnp.sum(out * a_ref[...][:, 0][None, :], axis=1, keepdims=True)
  alpha = jax.nn.sigmoid(t)
  hout_ref[...] = out * alpha
  alpha_ref[...] = alpha


def kernel(h, edge_index, W, b, a):
  n, d_in = h.shape
  d_out = W.shape[1]
  e = edge_index.shape[1]
  n_pad = -(-(n + 1) // (NS * BLK)) * (NS * BLK)

  ept = -(-e // NW)
  nfull = ept // BLK
  t = ept - nfull * BLK
  t8 = -(-t // 8) * 8
  cap = nfull * BLK + t8
  fill = NW * cap - e

  spare = jnp.arange(fill, dtype=jnp.int32) % (n_pad - n - 1)
  src_f = jnp.concatenate([edge_index[0], jnp.full((fill,), n, jnp.int32)])
  dst_f = jnp.concatenate([edge_index[1], n + 1 + spare])
  src_f = src_f.reshape(NW, cap)
  dst_f = dst_f.reshape(NW, cap)
  sbg = 2
  for _cand in range(2, 33, 2):
    if nfull % _cand == 0:
      sbg = _cand
  if nfull % 2 or nfull == 0:
    sbg = max(nfull, 1)
  src_m3 = src_f[:, :nfull * BLK].reshape(NW, nfull, BLK)
  dst_m3 = dst_f[:, :nfull * BLK].reshape(NW, nfull, BLK)
  src_m = src_m3.reshape(NW, max(nfull, 1) // sbg, sbg, BLK)
  dst_m = dst_m3.reshape(NW, max(nfull, 1) // sbg, sbg, BLK)
  if t8:
    src_t = src_f[:, nfull * BLK:]
    dst_t = dst_f[:, nfull * BLK:]
  else:
    src_t = jnp.zeros((NW, 8), jnp.int32)
    dst_t = jnp.full((NW, 8), n, jnp.int32)
  zdeg = jnp.zeros((n_pad,), jnp.float32)
  zrow = jnp.zeros((BLK, d_in), jnp.float32)

  deg_parts = _deg_call(nfull, t8, n_pad)(
      src_m3, dst_m3, src_t, dst_t, zdeg)

  grid = n_pad // 1024
  x = pl.pallas_call(
      _xnorm_body,
      grid=(grid,),
      in_specs=[
          pl.BlockSpec((NC, 2, 1024), lambda i: (0, 0, i)),
          pl.BlockSpec((1024, d_in), lambda i: (i, 0)),
      ],
      out_specs=pl.BlockSpec((1024, d_in), lambda i: (i, 0)),
      out_shape=jax.ShapeDtypeStruct((n_pad, d_in), jnp.float32),
  )(deg_parts, h)

  parts = _agg_call(nfull, t8, n_pad, d_in)(
      x, src_m, dst_m, src_t, dst_t, zrow)

  h_out, alpha = pl.pallas_call(
      _final_body,
      grid=(grid,),
      in_specs=[
          pl.BlockSpec((NC, 1024, d_in), lambda i: (0, i, 0)),
          pl.BlockSpec((NC, 2, 1024), lambda i: (0, 0, i)),
          pl.BlockSpec((d_in, d_out), lambda i: (0, 0)),
          pl.BlockSpec((d_out,), lambda i: (0,)),
          pl.BlockSpec((d_out, 1), lambda i: (0, 0)),
      ],
      out_specs=[
          pl.BlockSpec((1024, d_out), lambda i: (i, 0)),
          pl.BlockSpec((1024, 1), lambda i: (i, 0)),
      ],
      out_shape=[
          jax.ShapeDtypeStruct((n, d_out), jnp.float32),
          jax.ShapeDtypeStruct((n, 1), jnp.float32),
      ],
  )(parts, deg_parts, W, b, a)

  return (h_out, alpha)
